# Initial kernel scaffold; baseline (speedup 1.0000x reference)
import jax
import jax.numpy as jnp
from jax.experimental import pallas as pl


def kernel(x, router_w, w1, w2, w3):
    raise NotImplementedError("write your pallas kernel here")


# speedup vs baseline: 1.3290x; 1.3290x over previous
import functools

import jax
import jax.numpy as jnp
from jax import lax
from jax.experimental import pallas as pl
from jax.experimental.pallas import tpu as pltpu

NE = 8
DM = 768
DF = 3072
T = 2048
BM = 256
G = 23
R = G * BM
AUX_COEF = 0.01
Z_COEF = 0.001


def _router_body(x_ref, rw_ref, rows_ref, wp_ref, counts_ref, aux_ref, z_ref):
    x = x_ref[...]
    rw = rw_ref[...]
    l = lax.dot_general(x, rw, (((1,), (1,)), ((), ())),
                        preferred_element_type=jnp.float32)
    iota_e = lax.broadcasted_iota(jnp.int32, (T, NE), 1)
    m1 = jnp.max(l, axis=1, keepdims=True)
    i1 = jnp.min(jnp.where(l == m1, iota_e, NE), axis=1, keepdims=True)
    one1 = (iota_e == i1)
    lm = jnp.where(one1, -jnp.inf, l)
    m2 = jnp.max(lm, axis=1, keepdims=True)
    i2 = jnp.min(jnp.where(lm == m2, iota_e, NE), axis=1, keepdims=True)
    one2 = (iota_e == i2)
    w0 = 1.0 / (1.0 + jnp.exp(m2 - m1))
    w1v = 1.0 - w0
    A = one1.astype(jnp.float32) + one2.astype(jnp.float32)
    r_i = lax.broadcasted_iota(jnp.int32, (T, T), 0)
    c_i = lax.broadcasted_iota(jnp.int32, (T, T), 1)
    tri = (c_i < r_i).astype(jnp.float32)
    S = lax.dot_general(tri, A, (((1,), (0,)), ((), ())),
                        preferred_element_type=jnp.float32)
    counts = jnp.sum(A, axis=0, keepdims=True)
    pc = jnp.ceil(counts * (1.0 / BM)) * BM
    e_r = lax.broadcasted_iota(jnp.int32, (NE, NE), 0)
    e_c = lax.broadcasted_iota(jnp.int32, (NE, NE), 1)
    tri8 = (e_r < e_c).astype(jnp.float32)
    off = lax.dot_general(pc, tri8, (((1,), (0,)), ((), ())),
                          preferred_element_type=jnp.float32)
    off0 = jnp.sum(off * one1, axis=1, keepdims=True)
    off1 = jnp.sum(off * one2, axis=1, keepdims=True)
    pos0 = jnp.sum(S * one1, axis=1, keepdims=True)
    pos1 = jnp.sum(S * one2, axis=1, keepdims=True)
    row0 = (off0 + pos0).astype(jnp.int32)
    row1 = (off1 + pos1).astype(jnp.int32)
    rows_ref[...] = jnp.concatenate([row0, row1], axis=1)
    wp_ref[...] = jnp.concatenate([w0, w1v], axis=1)
    counts_ref[...] = counts
    p = jnp.exp(l - m1)
    p = p / jnp.sum(p, axis=1, keepdims=True)
    imp = jnp.mean(p, axis=0, keepdims=True)
    load = jnp.mean(one1.astype(jnp.float32), axis=0, keepdims=True)
    aux_ref[...] = (NE * AUX_COEF) * jnp.sum(imp * load, keepdims=True).reshape(1, 1)
    z_ref[...] = Z_COEF * jnp.mean(l * l, keepdims=True).reshape(1, 1)


def _router(x2, rw):
    return pl.pallas_call(
        _router_body,
        out_shape=[
            jax.ShapeDtypeStruct((T, 2), jnp.int32),
            jax.ShapeDtypeStruct((T, 2), jnp.float32),
            jax.ShapeDtypeStruct((1, NE), jnp.float32),
            jax.ShapeDtypeStruct((1, 1), jnp.float32),
            jax.ShapeDtypeStruct((1, 1), jnp.float32),
        ],
        interpret=False,
    )(x2, rw)


def _ffn_body(be_ref, act_ref, xg_ref, w1_ref, w3_ref, w2_ref, wr_ref, y_ref):
    j = pl.program_id(0)

    @pl.when(act_ref[j] > 0)
    def _():
        xb = xg_ref[...]
        g = lax.dot_general(xb, w1_ref[0], (((1,), (0,)), ((), ())),
                            preferred_element_type=jnp.float32)
        u = lax.dot_general(xb, w3_ref[0], (((1,), (0,)), ((), ())),
                            preferred_element_type=jnp.float32)
        h = (g * jax.nn.sigmoid(g)) * u
        y = lax.dot_general(h, w2_ref[0], (((1,), (0,)), ((), ())),
                            preferred_element_type=jnp.float32)
        y_ref[...] = y * wr_ref[...]


def _ffn(be, act, xg, w1, w3, w2, wr):
    grid_spec = pltpu.PrefetchScalarGridSpec(
        num_scalar_prefetch=2,
        grid=(G,),
        in_specs=[
            pl.BlockSpec((BM, DM), lambda j, be, act: (j, 0)),
            pl.BlockSpec((1, DM, DF), lambda j, be, act: (be[j], 0, 0)),
            pl.BlockSpec((1, DM, DF), lambda j, be, act: (be[j], 0, 0)),
            pl.BlockSpec((1, DF, DM), lambda j, be, act: (be[j], 0, 0)),
            pl.BlockSpec((BM, 1), lambda j, be, act: (j, 0)),
        ],
        out_specs=pl.BlockSpec((BM, DM), lambda j, be, act: (j, 0)),
    )
    return pl.pallas_call(
        _ffn_body,
        grid_spec=grid_spec,
        out_shape=jax.ShapeDtypeStruct((R, DM), jnp.float32),
        compiler_params=pltpu.CompilerParams(
            dimension_semantics=("arbitrary",),
            vmem_limit_bytes=100 * 1024 * 1024),
        interpret=False,
    )(be, act, xg, w1, w3, w2, wr)


def kernel(x, router_w, w1, w2, w3):
    b, s, d = x.shape
    x2 = x.reshape(s * b, d)
    rows, wp, counts, aux, z = _router(x2, router_w)
    r0 = rows[:, 0]
    r1 = rows[:, 1]
    c = counts[0]
    nb = jnp.ceil(c * (1.0 / BM)).astype(jnp.int32)
    nbc = jnp.cumsum(nb)
    total = nbc[-1]
    jj = jnp.arange(G, dtype=jnp.int32)
    act = (jj < total).astype(jnp.int32)
    jcl = jnp.minimum(jj, total - 1)
    be = jnp.sum((nbc[None, :] <= jcl[:, None]).astype(jnp.int32), axis=1)
    be = jnp.minimum(be, NE - 1)
    t_ids = jnp.arange(T, dtype=jnp.int32)
    row_tok = jnp.zeros((R,), jnp.int32).at[r0].set(t_ids).at[r1].set(t_ids)
    w_row = jnp.zeros((R,), jnp.float32).at[r0].set(wp[:, 0]).at[r1].set(wp[:, 1])
    xg = x2[row_tok]
    y = _ffn(be, act, xg, w1, w3, w2, w_row[:, None])
    out = y[r0] + y[r1]
    return out.reshape(b, s, d), aux.reshape(()), z.reshape(())
